# Initial kernel scaffold; baseline (speedup 1.0000x reference)
import jax
import jax.numpy as jnp
from jax.experimental import pallas as pl


def kernel(x, edge_index, edge_attr, batch, eps1, W1a, b1a, W1b, b1b, eps2, W2a, b2a, W2b, b2b, Wc, bc):
    raise NotImplementedError("write your pallas kernel here")


# speedup vs baseline: 3.2001x; 3.2001x over previous
import functools

import jax
import jax.numpy as jnp
from jax import lax
from jax.experimental import pallas as pl
from jax.experimental.pallas import tpu as pltpu
from jax.experimental.pallas import tpu_sc as plsc

N = 10000
E = 320000
D = 128
H = 128
C = 4
G = 64

NC = 2
NS = 16
NW = NC * NS
EPW = E // NW
B = 80
NCHUNK = EPW // B
ZR = 80
ZCH = N // ZR
ZPT = -(-ZCH // NS)


def _sc_aggregate(feats, src, dst, edge_attr):
  mesh = plsc.VectorSubcoreMesh(
      core_axis_name="c", subcore_axis_name="s",
      num_cores=NC, num_subcores=NS)

  @functools.partial(
      pl.kernel,
      out_type=jax.ShapeDtypeStruct((NC * N, D), jnp.float32),
      mesh=mesh,
      scratch_types=[
          pltpu.VMEM((B,), jnp.int32),
          pltpu.VMEM((B,), jnp.int32),
          pltpu.VMEM((B, D), jnp.float32),
          pltpu.VMEM((B, D), jnp.float32),
          pltpu.VMEM_SHARED((N, D), jnp.float32),
          pltpu.SemaphoreType.DMA,
      ],
  )
  def k(x_hbm, src_hbm, dst_hbm, ea_hbm, out_hbm,
        sidx, didx, ea, xg, acc, sem):
    cid = lax.axis_index("c")
    sid = lax.axis_index("s")
    wid = sid * NC + cid

    def zrow(i, carry):
      for j in range(D // 16):
        ea[i, pl.ds(j * 16, 16)] = jnp.zeros((16,), jnp.float32)
      return carry

    lax.fori_loop(0, ZR, zrow, 0)

    def zchunk(ci, carry):
      cg = sid * ZPT + ci

      @pl.when(cg < ZCH)
      def _():
        pltpu.sync_copy(ea, acc.at[pl.ds(cg * ZR, ZR)])

      return carry

    lax.fori_loop(0, ZPT, zchunk, 0)
    plsc.subcore_barrier()

    def chunk(ci, carry):
      base = wid * EPW + ci * B
      pltpu.sync_copy(src_hbm.at[pl.ds(base, B)], sidx)
      pltpu.sync_copy(dst_hbm.at[pl.ds(base, B)], didx)
      pltpu.sync_copy(ea_hbm.at[pl.ds(base, B)], ea)
      pltpu.async_copy(x_hbm.at[sidx], xg, sem).wait()

      def row(i, c2):
        for j in range(D // 16):
          s = pl.ds(j * 16, 16)
          ea[i, s] = jnp.maximum(ea[i, s] + xg[i, s], 0.0)
        return c2

      lax.fori_loop(0, B, row, 0)
      pltpu.sync_copy(ea, acc.at[didx], add=True)
      return carry

    lax.fori_loop(0, NCHUNK, chunk, 0)
    plsc.subcore_barrier()

    def drain(ci, carry):
      cg = sid * ZPT + ci

      @pl.when(cg < ZCH)
      def _():
        pltpu.sync_copy(acc.at[pl.ds(cg * ZR, ZR)], ea)
        pltpu.sync_copy(ea, out_hbm.at[pl.ds(cid * N + cg * ZR, ZR)])

      return carry

    lax.fori_loop(0, ZPT, drain, 0)

  return k(feats, src, dst, edge_attr)


R = 1000
NBLK = N // R


def _tc_mlp(x, agg, eps, W1, b1, W2, b2):
  def body(eps_ref, x_ref, a0_ref, a1_ref, w1_ref, b1_ref, w2_ref, b2_ref,
           o_ref):
    scale = 1.0 + eps_ref[0]
    u = scale * x_ref[...] + a0_ref[...] + a1_ref[...]
    h = jnp.dot(u, w1_ref[...], preferred_element_type=jnp.float32)
    h = jnp.maximum(h + b1_ref[...], 0.0)
    h = jnp.dot(h, w2_ref[...], preferred_element_type=jnp.float32)
    o_ref[...] = jnp.maximum(h + b2_ref[...], 0.0)

  return pl.pallas_call(
      body,
      grid=(NBLK,),
      in_specs=[
          pl.BlockSpec(memory_space=pltpu.SMEM),
          pl.BlockSpec((R, D), lambda i: (i, 0)),
          pl.BlockSpec((R, D), lambda i: (i, 0)),
          pl.BlockSpec((R, D), lambda i: (i + NBLK, 0)),
          pl.BlockSpec((D, H), lambda i: (0, 0)),
          pl.BlockSpec((1, H), lambda i: (0, 0)),
          pl.BlockSpec((H, H), lambda i: (0, 0)),
          pl.BlockSpec((1, H), lambda i: (0, 0)),
      ],
      out_specs=pl.BlockSpec((R, H), lambda i: (i, 0)),
      out_shape=jax.ShapeDtypeStruct((N, H), jnp.float32),
  )(eps.reshape(1), x, agg, agg, W1, b1.reshape(1, H), W2, b2.reshape(1, H))


def _tc_mlp_pool(x, agg, eps, W1, b1, W2, b2, batch_row):
  def body(eps_ref, x_ref, a0_ref, a1_ref, w1_ref, b1_ref, w2_ref, b2_ref,
           b_ref, sums_ref, cnts_ref):
    scale = 1.0 + eps_ref[0]
    u = scale * x_ref[...] + a0_ref[...] + a1_ref[...]
    h = jnp.dot(u, w1_ref[...], preferred_element_type=jnp.float32)
    h = jnp.maximum(h + b1_ref[...], 0.0)
    h = jnp.dot(h, w2_ref[...], preferred_element_type=jnp.float32)
    h = jnp.maximum(h + b2_ref[...], 0.0)
    gids = lax.broadcasted_iota(jnp.int32, (G, 1), 0)
    mask = (gids == b_ref[0]).astype(jnp.float32)
    ps = jnp.dot(mask, h, preferred_element_type=jnp.float32)
    pc = jnp.dot(mask, jnp.ones((R, H), jnp.float32),
                 preferred_element_type=jnp.float32)

    @pl.when(pl.program_id(0) == 0)
    def _():
      sums_ref[...] = jnp.zeros_like(sums_ref)
      cnts_ref[...] = jnp.zeros_like(cnts_ref)

    sums_ref[...] += ps
    cnts_ref[...] += pc

  return pl.pallas_call(
      body,
      grid=(NBLK,),
      in_specs=[
          pl.BlockSpec(memory_space=pltpu.SMEM),
          pl.BlockSpec((R, D), lambda i: (i, 0)),
          pl.BlockSpec((R, D), lambda i: (i, 0)),
          pl.BlockSpec((R, D), lambda i: (i + NBLK, 0)),
          pl.BlockSpec((D, H), lambda i: (0, 0)),
          pl.BlockSpec((1, H), lambda i: (0, 0)),
          pl.BlockSpec((H, H), lambda i: (0, 0)),
          pl.BlockSpec((1, H), lambda i: (0, 0)),
          pl.BlockSpec((1, 1, R), lambda i: (i, 0, 0)),
      ],
      out_specs=[
          pl.BlockSpec((G, H), lambda i: (0, 0)),
          pl.BlockSpec((G, H), lambda i: (0, 0)),
      ],
      out_shape=[
          jax.ShapeDtypeStruct((G, H), jnp.float32),
          jax.ShapeDtypeStruct((G, H), jnp.float32),
      ],
  )(eps.reshape(1), x, agg, agg, W1, b1.reshape(1, H), W2, b2.reshape(1, H),
    batch_row)


def _tc_head(sums, cnts, Wc_pad, bc_pad):
  def body(s_ref, c_ref, w_ref, b_ref, o_ref):
    pooled = s_ref[...] / jnp.maximum(c_ref[...], 1.0)
    o_ref[...] = jnp.dot(pooled, w_ref[...],
                         preferred_element_type=jnp.float32) + b_ref[...]

  return pl.pallas_call(
      body,
      out_shape=jax.ShapeDtypeStruct((G, 128), jnp.float32),
  )(sums, cnts, Wc_pad, bc_pad)


def kernel(x, edge_index, edge_attr, batch, eps1, W1a, b1a, W1b, b1b,
           eps2, W2a, b2a, W2b, b2b, Wc, bc):
  src = edge_index[0]
  dst = edge_index[1]
  batch_row = batch.reshape(NBLK, 1, R)
  Wc_pad = jnp.concatenate(
      [Wc, jnp.zeros((H, 128 - C), jnp.float32)], axis=1)
  bc_pad = jnp.concatenate(
      [bc, jnp.zeros((128 - C,), jnp.float32)]).reshape(1, 128)

  agg1 = _sc_aggregate(x, src, dst, edge_attr)
  h1 = _tc_mlp(x, agg1, eps1, W1a, b1a, W1b, b1b)
  agg2 = _sc_aggregate(h1, src, dst, edge_attr)
  sums, cnts = _tc_mlp_pool(h1, agg2, eps2, W2a, b2a, W2b, b2b, batch_row)
  logits = _tc_head(sums, cnts, Wc_pad, bc_pad)
  return logits[:, :C]


# speedup vs baseline: 5.1937x; 1.6230x over previous
import functools

import jax
import jax.numpy as jnp
from jax import lax
from jax.experimental import pallas as pl
from jax.experimental.pallas import tpu as pltpu
from jax.experimental.pallas import tpu_sc as plsc

N = 10000
E = 320000
D = 128
H = 128
C = 4
G = 64

NC = 2
NS = 16
NW = NC * NS
EPW = E // NW
B = 80
NCHUNK = EPW // B
ZR = 80
ZCH = N // ZR
ZPT = -(-ZCH // NS)


NBUF = 3


def _sc_aggregate(feats, src, dst, edge_attr):
  mesh = plsc.VectorSubcoreMesh(
      core_axis_name="c", subcore_axis_name="s",
      num_cores=NC, num_subcores=NS)

  @functools.partial(
      pl.kernel,
      out_type=jax.ShapeDtypeStruct((NC * N, D), jnp.float32),
      mesh=mesh,
      scratch_types=[
          [pltpu.VMEM((B,), jnp.int32) for _ in range(2)],
          [pltpu.VMEM((B,), jnp.int32) for _ in range(2)],
          [pltpu.VMEM((B, D), jnp.float32) for _ in range(2)],
          [pltpu.VMEM((B, D), jnp.float32) for _ in range(2)],
          pltpu.VMEM_SHARED((N, D), jnp.float32),
          pltpu.SemaphoreType.DMA,
          pltpu.SemaphoreType.DMA,
          pltpu.SemaphoreType.DMA,
      ],
  )
  def k(x_hbm, src_hbm, dst_hbm, ea_hbm, out_hbm,
        sidx, didx, eab, xgb, acc, sem_ea, sem_g, sem_sc):
    cid = lax.axis_index("c")
    sid = lax.axis_index("s")
    wid = sid * NC + cid

    def start_ea(c, b):
      pltpu.async_copy(ea_hbm.at[pl.ds(wid * EPW + c * B, B)], eab[b],
                       sem_ea)

    def wait_ea(c, b):
      pltpu.make_async_copy(ea_hbm.at[pl.ds(wid * EPW + c * B, B)],
                            eab[b], sem_ea).wait()

    def load_idx(c, b):
      base = wid * EPW + c * B
      pltpu.sync_copy(src_hbm.at[pl.ds(base, B)], sidx[b])
      pltpu.sync_copy(dst_hbm.at[pl.ds(base, B)], didx[b])

    def start_g(c, b):
      pltpu.async_copy(x_hbm.at[sidx[b]], xgb[b], sem_g)

    def wait_g(c, b):
      pltpu.make_async_copy(x_hbm.at[sidx[b]], xgb[b], sem_g).wait()

    def start_sc(c, b):
      pltpu.async_copy(eab[b], acc.at[didx[b]], sem_sc, add=True)

    def wait_sc(c, b):
      pltpu.make_async_copy(eab[b], acc.at[didx[b]], sem_sc).wait()

    def compute(c, b):
      def row(i, c2):
        for j in range(D // 16):
          sl = pl.ds(j * 16, 16)
          eab[b][i, sl] = jnp.maximum(eab[b][i, sl] + xgb[b][i, sl], 0.0)
        return c2

      lax.fori_loop(0, B, row, 0)

    def body(c, b, do_pref, do_wait_sc):
      b1 = 1 - b
      wait_ea(c, b)
      wait_g(c, b)
      if do_wait_sc:
        wait_sc(c - 1, b1)
      if do_pref:
        load_idx(c + 1, b1)
        start_ea(c + 1, b1)
        start_g(c + 1, b1)
      compute(c, b)
      start_sc(c, b)

    def zrow(i, carry):
      for j in range(D // 16):
        eab[0][i, pl.ds(j * 16, 16)] = jnp.zeros((16,), jnp.float32)
      return carry

    lax.fori_loop(0, ZR, zrow, 0)

    def zchunk(ci, carry):
      cg = sid * ZPT + ci

      @pl.when(cg < ZCH)
      def _():
        pltpu.sync_copy(eab[0], acc.at[pl.ds(cg * ZR, ZR)])

      return carry

    lax.fori_loop(0, ZPT, zchunk, 0)
    plsc.subcore_barrier()

    load_idx(0, 0)
    start_ea(0, 0)
    start_g(0, 0)
    body(0, 0, True, False)
    body(1, 1, True, True)

    def outer(co, carry):
      cb = co * 2
      body(cb, 0, True, True)
      body(cb + 1, 1, True, True)
      return carry

    lax.fori_loop(1, NCHUNK // 2, outer, 0)
    body(NCHUNK - 1, 0, False, True)
    wait_sc(NCHUNK - 1, 0)
    plsc.subcore_barrier()

    def drain(ci, carry):
      cg = sid * ZPT + ci

      @pl.when(cg < ZCH)
      def _():
        pltpu.sync_copy(acc.at[pl.ds(cg * ZR, ZR)], eab[0])
        pltpu.sync_copy(eab[0], out_hbm.at[pl.ds(cid * N + cg * ZR, ZR)])

      return carry

    lax.fori_loop(0, ZPT, drain, 0)

  return k(feats, src, dst, edge_attr)


R = 1000
NBLK = N // R


def _tc_mlp(x, agg, eps, W1, b1, W2, b2):
  def body(eps_ref, x_ref, a0_ref, a1_ref, w1_ref, b1_ref, w2_ref, b2_ref,
           o_ref):
    scale = 1.0 + eps_ref[0]
    u = scale * x_ref[...] + a0_ref[...] + a1_ref[...]
    h = jnp.dot(u, w1_ref[...], preferred_element_type=jnp.float32)
    h = jnp.maximum(h + b1_ref[...], 0.0)
    h = jnp.dot(h, w2_ref[...], preferred_element_type=jnp.float32)
    o_ref[...] = jnp.maximum(h + b2_ref[...], 0.0)

  return pl.pallas_call(
      body,
      grid=(NBLK,),
      in_specs=[
          pl.BlockSpec(memory_space=pltpu.SMEM),
          pl.BlockSpec((R, D), lambda i: (i, 0)),
          pl.BlockSpec((R, D), lambda i: (i, 0)),
          pl.BlockSpec((R, D), lambda i: (i + NBLK, 0)),
          pl.BlockSpec((D, H), lambda i: (0, 0)),
          pl.BlockSpec((1, H), lambda i: (0, 0)),
          pl.BlockSpec((H, H), lambda i: (0, 0)),
          pl.BlockSpec((1, H), lambda i: (0, 0)),
      ],
      out_specs=pl.BlockSpec((R, H), lambda i: (i, 0)),
      out_shape=jax.ShapeDtypeStruct((N, H), jnp.float32),
  )(eps.reshape(1), x, agg, agg, W1, b1.reshape(1, H), W2, b2.reshape(1, H))


def _tc_mlp_pool(x, agg, eps, W1, b1, W2, b2, batch_row):
  def body(eps_ref, x_ref, a0_ref, a1_ref, w1_ref, b1_ref, w2_ref, b2_ref,
           b_ref, sums_ref, cnts_ref):
    scale = 1.0 + eps_ref[0]
    u = scale * x_ref[...] + a0_ref[...] + a1_ref[...]
    h = jnp.dot(u, w1_ref[...], preferred_element_type=jnp.float32)
    h = jnp.maximum(h + b1_ref[...], 0.0)
    h = jnp.dot(h, w2_ref[...], preferred_element_type=jnp.float32)
    h = jnp.maximum(h + b2_ref[...], 0.0)
    gids = lax.broadcasted_iota(jnp.int32, (G, 1), 0)
    mask = (gids == b_ref[0]).astype(jnp.float32)
    ps = jnp.dot(mask, h, preferred_element_type=jnp.float32)
    pc = jnp.dot(mask, jnp.ones((R, H), jnp.float32),
                 preferred_element_type=jnp.float32)

    @pl.when(pl.program_id(0) == 0)
    def _():
      sums_ref[...] = jnp.zeros_like(sums_ref)
      cnts_ref[...] = jnp.zeros_like(cnts_ref)

    sums_ref[...] += ps
    cnts_ref[...] += pc

  return pl.pallas_call(
      body,
      grid=(NBLK,),
      in_specs=[
          pl.BlockSpec(memory_space=pltpu.SMEM),
          pl.BlockSpec((R, D), lambda i: (i, 0)),
          pl.BlockSpec((R, D), lambda i: (i, 0)),
          pl.BlockSpec((R, D), lambda i: (i + NBLK, 0)),
          pl.BlockSpec((D, H), lambda i: (0, 0)),
          pl.BlockSpec((1, H), lambda i: (0, 0)),
          pl.BlockSpec((H, H), lambda i: (0, 0)),
          pl.BlockSpec((1, H), lambda i: (0, 0)),
          pl.BlockSpec((1, 1, R), lambda i: (i, 0, 0)),
      ],
      out_specs=[
          pl.BlockSpec((G, H), lambda i: (0, 0)),
          pl.BlockSpec((G, H), lambda i: (0, 0)),
      ],
      out_shape=[
          jax.ShapeDtypeStruct((G, H), jnp.float32),
          jax.ShapeDtypeStruct((G, H), jnp.float32),
      ],
  )(eps.reshape(1), x, agg, agg, W1, b1.reshape(1, H), W2, b2.reshape(1, H),
    batch_row)


def _tc_head(sums, cnts, Wc_pad, bc_pad):
  def body(s_ref, c_ref, w_ref, b_ref, o_ref):
    pooled = s_ref[...] / jnp.maximum(c_ref[...], 1.0)
    o_ref[...] = jnp.dot(pooled, w_ref[...],
                         preferred_element_type=jnp.float32) + b_ref[...]

  return pl.pallas_call(
      body,
      out_shape=jax.ShapeDtypeStruct((G, 128), jnp.float32),
  )(sums, cnts, Wc_pad, bc_pad)


def kernel(x, edge_index, edge_attr, batch, eps1, W1a, b1a, W1b, b1b,
           eps2, W2a, b2a, W2b, b2b, Wc, bc):
  src1 = edge_index[0]
  dst1 = edge_index[1]
  batch_row = batch.reshape(NBLK, 1, R)
  Wc_pad = jnp.concatenate(
      [Wc, jnp.zeros((H, 128 - C), jnp.float32)], axis=1)
  bc_pad = jnp.concatenate(
      [bc, jnp.zeros((128 - C,), jnp.float32)]).reshape(1, 128)

  agg1 = _sc_aggregate(x, src1, dst1, edge_attr)
  h1 = _tc_mlp(x, agg1, eps1, W1a, b1a, W1b, b1b)
  agg2 = _sc_aggregate(h1, src1, dst1, edge_attr)
  sums, cnts = _tc_mlp_pool(h1, agg2, eps2, W2a, b2a, W2b, b2b, batch_row)
  logits = _tc_head(sums, cnts, Wc_pad, bc_pad)
  return logits[:, :C]


# speedup vs baseline: 5.2372x; 1.0084x over previous
import functools

import jax
import jax.numpy as jnp
from jax import lax
from jax.experimental import pallas as pl
from jax.experimental.pallas import tpu as pltpu
from jax.experimental.pallas import tpu_sc as plsc

N = 10000
E = 320000
D = 128
H = 128
C = 4
G = 64

NC = 2
NS = 16
NW = NC * NS
EPW = E // NW
B = 80
NCHUNK = EPW // B
ZR = 80
ZCH = N // ZR
ZPT = -(-ZCH // NS)


NBUF = 3


def _sc_aggregate(feats, src, dst, edge_attr):
  mesh = plsc.VectorSubcoreMesh(
      core_axis_name="c", subcore_axis_name="s",
      num_cores=NC, num_subcores=NS)

  @functools.partial(
      pl.kernel,
      out_type=jax.ShapeDtypeStruct((NC * N, D), jnp.float32),
      mesh=mesh,
      scratch_types=[
          [pltpu.VMEM((B,), jnp.int32) for _ in range(2)],
          [pltpu.VMEM((B,), jnp.int32) for _ in range(2)],
          [pltpu.VMEM((B, D), jnp.float32) for _ in range(2)],
          [pltpu.VMEM((B, D), jnp.float32) for _ in range(2)],
          pltpu.VMEM_SHARED((N, D), jnp.float32),
          pltpu.SemaphoreType.DMA,
          pltpu.SemaphoreType.DMA,
          pltpu.SemaphoreType.DMA,
      ],
  )
  def k(x_hbm, src_hbm, dst_hbm, ea_hbm, out_hbm,
        sidx, didx, eab, xgb, acc, sem_ea, sem_g, sem_sc):
    cid = lax.axis_index("c")
    sid = lax.axis_index("s")
    wid = sid * NC + cid

    def start_ea(c, b):
      pltpu.async_copy(ea_hbm.at[pl.ds(wid * EPW + c * B, B)], eab[b],
                       sem_ea)

    def wait_ea(c, b):
      pltpu.make_async_copy(ea_hbm.at[pl.ds(wid * EPW + c * B, B)],
                            eab[b], sem_ea).wait()

    def load_idx(c, b):
      base = wid * EPW + c * B
      pltpu.sync_copy(src_hbm.at[pl.ds(base, B)], sidx[b])
      pltpu.sync_copy(dst_hbm.at[pl.ds(base, B)], didx[b])

    def start_g(c, b):
      pltpu.async_copy(x_hbm.at[sidx[b]], xgb[b], sem_g)

    def wait_g(c, b):
      pltpu.make_async_copy(x_hbm.at[sidx[b]], xgb[b], sem_g).wait()

    def start_sc(c, b):
      pltpu.async_copy(eab[b], acc.at[didx[b]], sem_sc, add=True)

    def wait_sc(c, b):
      pltpu.make_async_copy(eab[b], acc.at[didx[b]], sem_sc).wait()

    def compute(c, b):
      def row(i2, c2):
        i = i2 * 2
        for u in range(2):
          for j in range(D // 16):
            sl = pl.ds(j * 16, 16)
            eab[b][i + u, sl] = jnp.maximum(
                eab[b][i + u, sl] + xgb[b][i + u, sl], 0.0)
        return c2

      lax.fori_loop(0, B // 2, row, 0)

    def body(c, b, do_pref, do_wait_sc):
      b1 = 1 - b
      wait_ea(c, b)
      wait_g(c, b)
      if do_wait_sc:
        wait_sc(c - 1, b1)
      if do_pref:
        load_idx(c + 1, b1)
        start_ea(c + 1, b1)
        start_g(c + 1, b1)
      compute(c, b)
      start_sc(c, b)

    def zrow(i, carry):
      for j in range(D // 16):
        eab[0][i, pl.ds(j * 16, 16)] = jnp.zeros((16,), jnp.float32)
      return carry

    lax.fori_loop(0, ZR, zrow, 0)

    def zchunk(ci, carry):
      cg = sid * ZPT + ci

      @pl.when(cg < ZCH)
      def _():
        pltpu.sync_copy(eab[0], acc.at[pl.ds(cg * ZR, ZR)])

      return carry

    lax.fori_loop(0, ZPT, zchunk, 0)
    plsc.subcore_barrier()

    load_idx(0, 0)
    start_ea(0, 0)
    start_g(0, 0)
    body(0, 0, True, False)
    body(1, 1, True, True)

    def outer(co, carry):
      cb = co * 2
      body(cb, 0, True, True)
      body(cb + 1, 1, True, True)
      return carry

    lax.fori_loop(1, NCHUNK // 2, outer, 0)
    body(NCHUNK - 1, 0, False, True)
    wait_sc(NCHUNK - 1, 0)
    plsc.subcore_barrier()

    def drain(ci, carry):
      cg = sid * ZPT + ci

      @pl.when(cg < ZCH)
      def _():
        pltpu.sync_copy(acc.at[pl.ds(cg * ZR, ZR)],
                        out_hbm.at[pl.ds(cid * N + cg * ZR, ZR)])

      return carry

    lax.fori_loop(0, ZPT, drain, 0)

  return k(feats, src, dst, edge_attr)


R = 1000
NBLK = N // R


def _tc_mlp(x, agg, eps, W1, b1, W2, b2):
  def body(eps_ref, x_ref, a0_ref, a1_ref, w1_ref, b1_ref, w2_ref, b2_ref,
           o_ref):
    scale = 1.0 + eps_ref[0]
    u = scale * x_ref[...] + a0_ref[...] + a1_ref[...]
    h = jnp.dot(u, w1_ref[...], preferred_element_type=jnp.float32)
    h = jnp.maximum(h + b1_ref[...], 0.0)
    h = jnp.dot(h, w2_ref[...], preferred_element_type=jnp.float32)
    o_ref[...] = jnp.maximum(h + b2_ref[...], 0.0)

  return pl.pallas_call(
      body,
      grid=(NBLK,),
      in_specs=[
          pl.BlockSpec(memory_space=pltpu.SMEM),
          pl.BlockSpec((R, D), lambda i: (i, 0)),
          pl.BlockSpec((R, D), lambda i: (i, 0)),
          pl.BlockSpec((R, D), lambda i: (i + NBLK, 0)),
          pl.BlockSpec((D, H), lambda i: (0, 0)),
          pl.BlockSpec((1, H), lambda i: (0, 0)),
          pl.BlockSpec((H, H), lambda i: (0, 0)),
          pl.BlockSpec((1, H), lambda i: (0, 0)),
      ],
      out_specs=pl.BlockSpec((R, H), lambda i: (i, 0)),
      out_shape=jax.ShapeDtypeStruct((N, H), jnp.float32),
  )(eps.reshape(1), x, agg, agg, W1, b1.reshape(1, H), W2, b2.reshape(1, H))


def _tc_mlp_pool(x, agg, eps, W1, b1, W2, b2, batch_row):
  def body(eps_ref, x_ref, a0_ref, a1_ref, w1_ref, b1_ref, w2_ref, b2_ref,
           b_ref, sums_ref, cnts_ref):
    scale = 1.0 + eps_ref[0]
    u = scale * x_ref[...] + a0_ref[...] + a1_ref[...]
    h = jnp.dot(u, w1_ref[...], preferred_element_type=jnp.float32)
    h = jnp.maximum(h + b1_ref[...], 0.0)
    h = jnp.dot(h, w2_ref[...], preferred_element_type=jnp.float32)
    h = jnp.maximum(h + b2_ref[...], 0.0)
    gids = lax.broadcasted_iota(jnp.int32, (G, 1), 0)
    mask = (gids == b_ref[0]).astype(jnp.float32)
    ps = jnp.dot(mask, h, preferred_element_type=jnp.float32)
    pc = jnp.dot(mask, jnp.ones((R, H), jnp.float32),
                 preferred_element_type=jnp.float32)

    @pl.when(pl.program_id(0) == 0)
    def _():
      sums_ref[...] = jnp.zeros_like(sums_ref)
      cnts_ref[...] = jnp.zeros_like(cnts_ref)

    sums_ref[...] += ps
    cnts_ref[...] += pc

  return pl.pallas_call(
      body,
      grid=(NBLK,),
      in_specs=[
          pl.BlockSpec(memory_space=pltpu.SMEM),
          pl.BlockSpec((R, D), lambda i: (i, 0)),
          pl.BlockSpec((R, D), lambda i: (i, 0)),
          pl.BlockSpec((R, D), lambda i: (i + NBLK, 0)),
          pl.BlockSpec((D, H), lambda i: (0, 0)),
          pl.BlockSpec((1, H), lambda i: (0, 0)),
          pl.BlockSpec((H, H), lambda i: (0, 0)),
          pl.BlockSpec((1, H), lambda i: (0, 0)),
          pl.BlockSpec((1, 1, R), lambda i: (i, 0, 0)),
      ],
      out_specs=[
          pl.BlockSpec((G, H), lambda i: (0, 0)),
          pl.BlockSpec((G, H), lambda i: (0, 0)),
      ],
      out_shape=[
          jax.ShapeDtypeStruct((G, H), jnp.float32),
          jax.ShapeDtypeStruct((G, H), jnp.float32),
      ],
  )(eps.reshape(1), x, agg, agg, W1, b1.reshape(1, H), W2, b2.reshape(1, H),
    batch_row)


def _tc_head(sums, cnts, Wc_pad, bc_pad):
  def body(s_ref, c_ref, w_ref, b_ref, o_ref):
    pooled = s_ref[...] / jnp.maximum(c_ref[...], 1.0)
    o_ref[...] = jnp.dot(pooled, w_ref[...],
                         preferred_element_type=jnp.float32) + b_ref[...]

  return pl.pallas_call(
      body,
      out_shape=jax.ShapeDtypeStruct((G, 128), jnp.float32),
  )(sums, cnts, Wc_pad, bc_pad)


def kernel(x, edge_index, edge_attr, batch, eps1, W1a, b1a, W1b, b1b,
           eps2, W2a, b2a, W2b, b2b, Wc, bc):
  src1 = edge_index[0]
  dst1 = edge_index[1]
  batch_row = batch.reshape(NBLK, 1, R)
  Wc_pad = jnp.concatenate(
      [Wc, jnp.zeros((H, 128 - C), jnp.float32)], axis=1)
  bc_pad = jnp.concatenate(
      [bc, jnp.zeros((128 - C,), jnp.float32)]).reshape(1, 128)

  agg1 = _sc_aggregate(x, src1, dst1, edge_attr)
  h1 = _tc_mlp(x, agg1, eps1, W1a, b1a, W1b, b1b)
  agg2 = _sc_aggregate(h1, src1, dst1, edge_attr)
  sums, cnts = _tc_mlp_pool(h1, agg2, eps2, W2a, b2a, W2b, b2b, batch_row)
  logits = _tc_head(sums, cnts, Wc_pad, bc_pad)
  return logits[:, :C]
